# baseline (speedup 1.0000x reference)
import functools

import jax
import jax.numpy as jnp
from jax import lax
from jax.experimental import pallas as pl
from jax.experimental.pallas import tpu as pltpu
from jax.experimental.pallas import tpu_sc as plsc

BATCH = 16384
HIDDEN = 64
OUT_DIM = 64

_NC = 2
_NS = 16
_NW = _NC * _NS
_BPW = BATCH // _NW
_CHUNK = 128
_NCHUNK = _BPW // _CHUNK


def _gather_body(idx_hbm, table_hbm, out_hbm, idx_v, rows_v, sem):
    wid = lax.axis_index("s") * _NC + lax.axis_index("c")
    base = wid * _BPW
    pltpu.sync_copy(idx_hbm.at[wid], idx_v)
    copies = []
    for j in range(_NCHUNK):
        copies.append(
            pltpu.async_copy(
                table_hbm.at[idx_v.at[j]],
                rows_v.at[pl.ds(j * _CHUNK, _CHUNK)],
                sem,
            )
        )
    for cp in copies:
        cp.wait()
    pltpu.sync_copy(rows_v, out_hbm.at[pl.ds(base, _BPW)])


_gather = pl.kernel(
    _gather_body,
    out_type=jax.ShapeDtypeStruct((BATCH, HIDDEN), jnp.float32),
    mesh=plsc.VectorSubcoreMesh(core_axis_name="c", subcore_axis_name="s"),
    scratch_types=[
        pltpu.VMEM((_NCHUNK, _CHUNK), jnp.int32),
        pltpu.VMEM((_BPW, HIDDEN), jnp.float32),
        pltpu.SemaphoreType.DMA,
    ],
    compiler_params=pltpu.CompilerParams(use_tc_tiling_on_sc=False),
)

_BM = 1024


def _proj_body(x_ref, w_ref, b_ref, o_ref):
    o_ref[...] = (
        lax.dot_general(
            x_ref[...],
            w_ref[...],
            (((1,), (1,)), ((), ())),
            preferred_element_type=jnp.float32,
        )
        + b_ref[...]
    )


_project = pl.pallas_call(
    _proj_body,
    grid=(BATCH // _BM,),
    in_specs=[
        pl.BlockSpec((_BM, HIDDEN), lambda i: (i, 0)),
        pl.BlockSpec((OUT_DIM, HIDDEN), lambda i: (0, 0)),
        pl.BlockSpec((1, OUT_DIM), lambda i: (0, 0)),
    ],
    out_specs=pl.BlockSpec((_BM, OUT_DIM), lambda i: (i, 0)),
    out_shape=jax.ShapeDtypeStruct((BATCH, OUT_DIM), jnp.float32),
)


def kernel(course_id, embed_table, W, b):
    idx = course_id.reshape(_NW, _NCHUNK, _CHUNK).astype(jnp.int32)
    rows = _gather(idx, embed_table)
    out = _project(rows, W, b.reshape(1, OUT_DIM))
    return out.reshape(BATCH, 1, OUT_DIM)


# speedup vs baseline: 2.4856x; 2.4856x over previous
import functools

import jax
import jax.numpy as jnp
from jax import lax
from jax.experimental import pallas as pl
from jax.experimental.pallas import tpu as pltpu
from jax.experimental.pallas import tpu_sc as plsc

BATCH = 16384
HIDDEN = 64
OUT_DIM = 64

_NC = 2
_NS = 16
_NW = _NC * _NS
_BPW = BATCH // _NW


def _gather_body(idx_hbm, table_hbm, out_hbm, idx_v, rows_v, sem):
    wid = lax.axis_index("s") * _NC + lax.axis_index("c")
    base = wid * _BPW
    pltpu.sync_copy(idx_hbm.at[pl.ds(base, _BPW)], idx_v)

    def body(g, _):
        v16 = idx_v[pl.ds(g * 16, 16)]
        for l in range(16):
            v = v16[l]
            pltpu.async_copy(
                table_hbm.at[v >> 3, v & 7], rows_v.at[g * 16 + l], sem)
        return 0

    lax.fori_loop(0, _BPW // 16, body, 0)
    pltpu.make_async_copy(out_hbm.at[pl.ds(base, _BPW)], rows_v, sem).wait()
    pltpu.sync_copy(rows_v, out_hbm.at[pl.ds(base, _BPW)])


_gather = pl.kernel(
    _gather_body,
    out_type=jax.ShapeDtypeStruct((BATCH, HIDDEN), jnp.float32),
    mesh=plsc.VectorSubcoreMesh(core_axis_name="c", subcore_axis_name="s"),
    scratch_types=[
        pltpu.VMEM((_BPW,), jnp.int32),
        pltpu.VMEM((_BPW, HIDDEN), jnp.float32),
        pltpu.SemaphoreType.DMA,
    ],
)

_BM = 1024


def _proj_body(x_ref, w_ref, b_ref, o_ref):
    o_ref[...] = (
        lax.dot_general(
            x_ref[...],
            w_ref[...],
            (((1,), (1,)), ((), ())),
            preferred_element_type=jnp.float32,
        )
        + b_ref[...]
    )


_project = pl.pallas_call(
    _proj_body,
    grid=(BATCH // _BM,),
    in_specs=[
        pl.BlockSpec((_BM, HIDDEN), lambda i: (i, 0)),
        pl.BlockSpec((OUT_DIM, HIDDEN), lambda i: (0, 0)),
        pl.BlockSpec((1, OUT_DIM), lambda i: (0, 0)),
    ],
    out_specs=pl.BlockSpec((_BM, OUT_DIM), lambda i: (i, 0)),
    out_shape=jax.ShapeDtypeStruct((BATCH, OUT_DIM), jnp.float32),
)


def kernel(course_id, embed_table, W, b):
    idx = course_id.reshape(BATCH).astype(jnp.int32)
    table3 = embed_table.reshape(125000, 8, HIDDEN)
    rows = _gather(idx, table3)
    out = _project(rows, W, b.reshape(1, OUT_DIM))
    return out.reshape(BATCH, 1, OUT_DIM)


# speedup vs baseline: 2.6036x; 1.0475x over previous
import functools

import jax
import jax.numpy as jnp
from jax import lax
from jax.experimental import pallas as pl
from jax.experimental.pallas import tpu as pltpu
from jax.experimental.pallas import tpu_sc as plsc

BATCH = 16384
HIDDEN = 64
OUT_DIM = 64

_NC = 2
_NS = 16
_NW = _NC * _NS
_BPW = BATCH // _NW


def _gather_body(idx_hbm, table_hbm, out_hbm, idx_v, rows_v, sem):
    wid = lax.axis_index("s") * _NC + lax.axis_index("c")
    base = wid * _BPW
    pltpu.sync_copy(idx_hbm.at[pl.ds(base, _BPW)], idx_v)

    def body(g, _):
        v16 = idx_v[pl.ds(g * 16, 16)]
        for l in range(16):
            v = v16[l]
            pltpu.async_copy(
                table_hbm.at[v >> 3, v & 7], rows_v.at[g * 16 + l], sem)
        return 0

    lax.fori_loop(0, _BPW // 16, body, 0)
    pltpu.make_async_copy(out_hbm.at[pl.ds(base, _BPW)], rows_v, sem).wait()
    pltpu.sync_copy(rows_v, out_hbm.at[pl.ds(base, _BPW)])


_gather = pl.kernel(
    _gather_body,
    out_type=jax.ShapeDtypeStruct((BATCH, HIDDEN), jnp.float32),
    mesh=plsc.VectorSubcoreMesh(core_axis_name="c", subcore_axis_name="s"),
    scratch_types=[
        pltpu.VMEM((_BPW,), jnp.int32),
        pltpu.VMEM((_BPW, HIDDEN), jnp.float32),
        pltpu.SemaphoreType.DMA,
    ],
)

_BM = 2048


def _proj_body(x_ref, w_ref, b_ref, o_ref):
    o_ref[...] = (
        lax.dot_general(
            w_ref[...],
            x_ref[...],
            (((1,), (1,)), ((), ())),
            preferred_element_type=jnp.float32,
        )
        + b_ref[...]
    )


_project = pl.pallas_call(
    _proj_body,
    grid=(BATCH // _BM,),
    in_specs=[
        pl.BlockSpec((_BM, HIDDEN), lambda i: (i, 0)),
        pl.BlockSpec((OUT_DIM, HIDDEN), lambda i: (0, 0)),
        pl.BlockSpec((OUT_DIM, 1), lambda i: (0, 0)),
    ],
    out_specs=pl.BlockSpec((OUT_DIM, _BM), lambda i: (0, i)),
    out_shape=jax.ShapeDtypeStruct((OUT_DIM, BATCH), jnp.float32),
)


def kernel(course_id, embed_table, W, b):
    idx = course_id.reshape(BATCH).astype(jnp.int32)
    table3 = embed_table.reshape(125000, 8, HIDDEN)
    rows = _gather(idx, table3)
    y = _project(rows, W, b.reshape(OUT_DIM, 1))
    return y.T.reshape(BATCH, 1, OUT_DIM)


# speedup vs baseline: 2.9929x; 1.1495x over previous
import functools

import jax
import jax.numpy as jnp
from jax import lax
from jax.experimental import pallas as pl
from jax.experimental.pallas import tpu as pltpu
from jax.experimental.pallas import tpu_sc as plsc

BATCH = 16384
HIDDEN = 64
OUT_DIM = 64

_NC = 2
_NS = 16
_NW = _NC * _NS
_NB = 7813
_CAP = 768
_ICH = 2048
_SENT = 500 << 21
_PAD = BATCH + 64


def _gather_body(idx_hbm, tableT_hbm, out_hbm,
                 idx_c, selk_v, band0, band1, rows_v,
                 off_s, gk_s, nb_s,
                 semA, semB, sem_out):
    wid = lax.axis_index("s") * _NC + lax.axis_index("c")
    lo = (_NB * wid) // _NW
    hi = (_NB * (wid + 1)) // _NW
    nloc = hi - lo
    iota = lax.iota(jnp.int32, 16)

    def sent(i, _):
        selk_v[pl.ds(i * 16, 16)] = jnp.full((16,), _SENT, jnp.int32)
        return 0
    lax.fori_loop(0, _CAP // 16, sent, 0)

    cnt = 0
    for ch in range(BATCH // _ICH):
        pltpu.sync_copy(idx_hbm.at[pl.ds(ch * _ICH, _ICH)], idx_c)

        def grp(g, cnt, ch=ch):
            v16 = idx_c[pl.ds(g * 16, 16)]
            rt16 = v16 >> 7
            lo16 = jnp.full((16,), lo, jnp.int32)
            hi16 = jnp.full((16,), hi, jnp.int32)
            one16 = jnp.full((16,), 1, jnp.int32)
            w16 = (rt16 - lo16) | (hi16 - one16 - rt16)
            mi = one16 - lax.shift_right_logical(w16, 31)
            c16 = plsc.cumsum(mi)
            k16 = (((rt16 - lo16) << 21) | ((v16 & 127) << 14)
                   | (iota + (ch * _ICH + g * 16)))
            base16 = jnp.full((16,), cnt, jnp.int32)
            dump16 = jnp.full((16,), _CAP + 8, jnp.int32)
            pos16 = mi * (base16 + c16 - one16) + (one16 - mi) * dump16
            plsc.store_scatter(selk_v, [pos16], k16)
            return cnt + c16[15]

        cnt = lax.fori_loop(0, _ICH // 16, grp, cnt)

    def zero(b, _):
        off_s[b] = 0
        return 0
    lax.fori_loop(0, 256, zero, 0)

    def count(g, _):
        k16 = selk_v[pl.ds(g * 16, 16)]
        for l in range(16):
            b = k16[l] >> 21
            bb = jnp.where(b < nloc, b, 255)
            off_s[bb] = off_s[bb] + 1
        return 0
    lax.fori_loop(0, _CAP // 16, count, 0)

    def prefix(b, carry):
        running, nb_cnt = carry
        c = off_s[b]
        use = b < nloc

        @pl.when(use & (c > 0))
        def _():
            nb_s[nb_cnt] = b
        running = jnp.where(use, running + c, running)
        nb_cnt = jnp.where(use & (c > 0), nb_cnt + 1, nb_cnt)

        @pl.when(use)
        def _():
            off_s[b] = running
        return running, nb_cnt

    total, m_nb = lax.fori_loop(0, 246, prefix, (0, 0))
    off_s[246] = total

    def scat(j, _):
        g = (_CAP // 16 - 1) - j
        k16 = selk_v[pl.ds(g * 16, 16)]
        for l in reversed(range(16)):
            k = k16[l]
            b = k >> 21

            @pl.when(b < nloc)
            def _(k=k, b=b):
                slot = off_s[b] - 1
                off_s[b] = slot
                gk_s[slot] = k
        return 0
    lax.fori_loop(0, _CAP // 16, scat, 0)

    def fetch(b_loc, buf, sem):
        return pltpu.async_copy(
            tableT_hbm.at[:, pl.ds((lo + b_loc) * 128, 128)], buf, sem)

    @pl.when(m_nb > 0)
    def _():
        fetch(nb_s[0], band0, semA)

    @pl.when(m_nb > 1)
    def _():
        fetch(nb_s[1], band1, semB)

    def pair(u, _):
        for par, buf, sem in ((0, band0, semA), (1, band1, semB)):
            t = 2 * u + par

            @pl.when(t < m_nb)
            def _(t=t, buf=buf, sem=sem):
                pltpu.make_async_copy(
                    tableT_hbm.at[:, pl.ds(0, 128)], buf, sem).wait()
                b = nb_s[t]
                start = off_s[b]
                end = off_s[jnp.where(b + 1 < nloc, b + 1, 246)]
                end = jnp.where(b + 1 < nloc, end, total)

                def extract(e, _):
                    k = gk_s[e]
                    lane = (k >> 14) & 127
                    for c in range(HIDDEN // 16):
                        vals = plsc.load_gather(
                            buf, [iota + 16 * c, jnp.full((16,), lane,
                                                          jnp.int32)])
                        plsc.store_scatter(
                            rows_v,
                            [jnp.full((16,), e, jnp.int32), iota + 16 * c],
                            vals)
                    return 0

                lax.fori_loop(start, end, extract, 0)

                @pl.when(t + 2 < m_nb)
                def _():
                    fetch(nb_s[t + 2], buf, sem)
        return 0

    lax.fori_loop(0, (246 + 1) // 2, pair, 0)

    def flush(g, _):
        for l in range(16):
            e = g * 16 + l
            real = e < total
            pos = jnp.where(real, gk_s[jnp.where(real, e, 0)] & 16383,
                            BATCH + wid)
            pltpu.async_copy(rows_v.at[e], out_hbm.at[pos], sem_out)
        return 0

    lax.fori_loop(0, _CAP // 16, flush, 0)
    pltpu.make_async_copy(out_hbm.at[pl.ds(0, _CAP)], rows_v, sem_out).wait()


_gather = pl.kernel(
    _gather_body,
    out_type=jax.ShapeDtypeStruct((_PAD, HIDDEN), jnp.float32),
    mesh=plsc.VectorSubcoreMesh(core_axis_name="c", subcore_axis_name="s"),
    scratch_types=[
        pltpu.VMEM((_ICH,), jnp.int32),
        pltpu.VMEM((_CAP + 32,), jnp.int32),
        pltpu.VMEM((HIDDEN, 128), jnp.float32),
        pltpu.VMEM((HIDDEN, 128), jnp.float32),
        pltpu.VMEM((_CAP, HIDDEN), jnp.float32),
        pltpu.SMEM((256,), jnp.int32),
        pltpu.SMEM((_CAP,), jnp.int32),
        pltpu.SMEM((256,), jnp.int32),
        pltpu.SemaphoreType.DMA,
        pltpu.SemaphoreType.DMA,
        pltpu.SemaphoreType.DMA,
    ],
    compiler_params=pltpu.CompilerParams(needs_layout_passes=False),
)

_BM = 2048


def _proj_body(x_ref, w_ref, b_ref, o_ref):
    o_ref[...] = (
        lax.dot_general(
            w_ref[...],
            x_ref[...],
            (((1,), (1,)), ((), ())),
            preferred_element_type=jnp.float32,
        )
        + b_ref[...]
    )


_project = pl.pallas_call(
    _proj_body,
    grid=(BATCH // _BM,),
    in_specs=[
        pl.BlockSpec((_BM, HIDDEN), lambda i: (i, 0)),
        pl.BlockSpec((OUT_DIM, HIDDEN), lambda i: (0, 0)),
        pl.BlockSpec((OUT_DIM, 1), lambda i: (0, 0)),
    ],
    out_specs=pl.BlockSpec((OUT_DIM, _BM), lambda i: (0, i)),
    out_shape=jax.ShapeDtypeStruct((OUT_DIM, BATCH), jnp.float32),
)


def kernel(course_id, embed_table, W, b):
    idx = course_id.reshape(BATCH).astype(jnp.int32)
    tableT = embed_table.T
    rows = _gather(idx, tableT)
    y = _project(rows, W, b.reshape(OUT_DIM, 1))
    return y.T.reshape(BATCH, 1, OUT_DIM)


# speedup vs baseline: 3.9446x; 1.3180x over previous
import functools

import jax
import jax.numpy as jnp
from jax import lax
from jax.experimental import pallas as pl
from jax.experimental.pallas import tpu as pltpu
from jax.experimental.pallas import tpu_sc as plsc

BATCH = 16384
HIDDEN = 64
OUT_DIM = 64

_NC = 2
_NS = 16
_NW = _NC * _NS
_NB = 7813
_CAP = 640
_ICH = 2048
_SENT = 500 << 21
_NBUF = 4
_PAD = BATCH + 64


def _gather_body(idx_hbm, tableT_hbm, out_hbm,
                 idx_c, selk_v, bands, rows_v,
                 off_s, gk_s, nb_s, sems, sem_out):
    wid = lax.axis_index("s") * _NC + lax.axis_index("c")
    lo = (_NB * wid) // _NW
    hi = (_NB * (wid + 1)) // _NW
    nloc = hi - lo
    iota = lax.iota(jnp.int32, 16)

    def round_body(carry):
        round_lo, _ = carry

        def sent(i, _):
            selk_v[pl.ds(i * 16, 16)] = jnp.full((16,), _SENT, jnp.int32)
            return 0
        lax.fori_loop(0, _CAP // 16, sent, 0)

        cnt = 0
        for ch in range(BATCH // _ICH):
            pltpu.sync_copy(idx_hbm.at[pl.ds(ch * _ICH, _ICH)], idx_c)

            def grp(g, cnt, ch=ch):
                v16 = idx_c[pl.ds(g * 16, 16)]
                rt16 = v16 >> 7
                lo16 = jnp.full((16,), lo, jnp.int32)
                hi16 = jnp.full((16,), hi, jnp.int32)
                one16 = jnp.full((16,), 1, jnp.int32)
                w16 = (rt16 - lo16) | (hi16 - one16 - rt16)
                mi = one16 - lax.shift_right_logical(w16, 31)
                c16 = plsc.cumsum(mi)
                ord16 = jnp.full((16,), cnt, jnp.int32) + c16 - one16
                rl16 = jnp.full((16,), round_lo, jnp.int32)
                cap16 = jnp.full((16,), _CAP, jnp.int32)
                win16 = (ord16 - rl16) | (rl16 + cap16 - one16 - ord16)
                mw = mi * (one16 - lax.shift_right_logical(win16, 31))
                k16 = (((rt16 - lo16) << 21) | ((v16 & 127) << 14)
                       | (iota + (ch * _ICH + g * 16)))
                dump16 = jnp.full((16,), _CAP + 8, jnp.int32)
                pos16 = (mw * (ord16 - rl16)
                         + (one16 - mw) * dump16)
                plsc.store_scatter(selk_v, [pos16], k16)
                return cnt + c16[15]

            cnt = lax.fori_loop(0, _ICH // 16, grp, cnt)

        def zero(b, _):
            off_s[b] = 0
            return 0
        lax.fori_loop(0, 256, zero, 0)

        def count(g, _):
            k16 = selk_v[pl.ds(g * 16, 16)]
            for l in range(16):
                b = k16[l] >> 21
                bb = jnp.where(b < nloc, b, 255)
                off_s[bb] = off_s[bb] + 1
            return 0
        lax.fori_loop(0, _CAP // 16, count, 0)

        def prefix(b, carry2):
            running, nb_cnt = carry2
            c = off_s[b]
            use = b < nloc

            @pl.when(use & (c > 0))
            def _():
                nb_s[nb_cnt] = b
            running = jnp.where(use, running + c, running)
            nb_cnt = jnp.where(use & (c > 0), nb_cnt + 1, nb_cnt)

            @pl.when(use)
            def _():
                off_s[b] = running
            return running, nb_cnt

        total, m_nb = lax.fori_loop(0, 246, prefix, (0, 0))
        off_s[246] = total

        def scat(j, _):
            g = (_CAP // 16 - 1) - j
            k16 = selk_v[pl.ds(g * 16, 16)]
            for l in reversed(range(16)):
                k = k16[l]
                b = k >> 21

                @pl.when(b < nloc)
                def _(k=k, b=b):
                    slot = off_s[b] - 1
                    off_s[b] = slot
                    gk_s[slot] = k
            return 0
        lax.fori_loop(0, _CAP // 16, scat, 0)

        def fetch(b_loc, buf, sem):
            pltpu.async_copy(
                tableT_hbm.at[:, pl.ds((lo + b_loc) * 128, 128)], buf, sem)

        for par in range(_NBUF):
            @pl.when(m_nb > par)
            def _(par=par):
                fetch(nb_s[par], bands[par], sems[par])

        def quad(u, _):
            for par in range(_NBUF):
                t = _NBUF * u + par
                buf = bands[par]
                sem = sems[par]

                @pl.when(t < m_nb)
                def _(t=t, buf=buf, sem=sem):
                    pltpu.make_async_copy(
                        tableT_hbm.at[:, pl.ds(0, 128)], buf, sem).wait()
                    b = nb_s[t]
                    start = off_s[b]
                    end = off_s[jnp.where(b + 1 < nloc, b + 1, 246)]
                    end = jnp.where(b + 1 < nloc, end, total)

                    def extract(e, _):
                        k = gk_s[e]
                        lane = (k >> 14) & 127
                        for c in range(HIDDEN // 16):
                            vals = plsc.load_gather(
                                buf, [iota + 16 * c,
                                      jnp.full((16,), lane, jnp.int32)])
                            plsc.store_scatter(
                                rows_v,
                                [jnp.full((16,), e, jnp.int32),
                                 iota + 16 * c],
                                vals)
                        return 0

                    lax.fori_loop(start, end, extract, 0)

                    @pl.when(t + _NBUF < m_nb)
                    def _():
                        fetch(nb_s[t + _NBUF], buf, sem)
            return 0

        lax.fori_loop(0, (246 + _NBUF - 1) // _NBUF, quad, 0)

        def flush(g, _):
            for l in range(16):
                e = g * 16 + l
                real = e < total
                pos = jnp.where(real, gk_s[jnp.where(real, e, 0)] & 16383,
                                BATCH + wid)
                pltpu.async_copy(rows_v.at[e], out_hbm.at[pos], sem_out)
            return 0

        lax.fori_loop(0, _CAP // 16, flush, 0)
        pltpu.make_async_copy(
            out_hbm.at[pl.ds(0, _CAP)], rows_v, sem_out).wait()

        return round_lo + _CAP, cnt

    lax.while_loop(lambda c: c[0] < c[1], round_body, (0, 1))


_gather = pl.kernel(
    _gather_body,
    out_type=jax.ShapeDtypeStruct((_PAD, HIDDEN), jnp.float32),
    mesh=plsc.VectorSubcoreMesh(core_axis_name="c", subcore_axis_name="s"),
    scratch_types=[
        pltpu.VMEM((_ICH,), jnp.int32),
        pltpu.VMEM((_CAP + 32,), jnp.int32),
        [pltpu.VMEM((HIDDEN, 128), jnp.float32) for _ in range(_NBUF)],
        pltpu.VMEM((_CAP, HIDDEN), jnp.float32),
        pltpu.SMEM((256,), jnp.int32),
        pltpu.SMEM((_CAP,), jnp.int32),
        pltpu.SMEM((256,), jnp.int32),
        [pltpu.SemaphoreType.DMA for _ in range(_NBUF)],
        pltpu.SemaphoreType.DMA,
    ],
    compiler_params=pltpu.CompilerParams(needs_layout_passes=False),
)

_BM = 2048


def _proj_body(x_ref, w_ref, b_ref, o_ref):
    o_ref[...] = (
        lax.dot_general(
            w_ref[...],
            x_ref[...],
            (((1,), (1,)), ((), ())),
            preferred_element_type=jnp.float32,
        )
        + b_ref[...]
    )


_project = pl.pallas_call(
    _proj_body,
    grid=(BATCH // _BM,),
    in_specs=[
        pl.BlockSpec((_BM, HIDDEN), lambda i: (i, 0)),
        pl.BlockSpec((OUT_DIM, HIDDEN), lambda i: (0, 0)),
        pl.BlockSpec((OUT_DIM, 1), lambda i: (0, 0)),
    ],
    out_specs=pl.BlockSpec((OUT_DIM, _BM), lambda i: (0, i)),
    out_shape=jax.ShapeDtypeStruct((OUT_DIM, BATCH), jnp.float32),
)


def kernel(course_id, embed_table, W, b):
    idx = course_id.reshape(BATCH).astype(jnp.int32)
    tableT = embed_table.T
    rows = _gather(idx, tableT)
    y = _project(rows, W, b.reshape(OUT_DIM, 1))
    return y.T.reshape(BATCH, 1, OUT_DIM)


# speedup vs baseline: 4.2263x; 1.0714x over previous
import functools

import jax
import jax.numpy as jnp
from jax import lax
from jax.experimental import pallas as pl
from jax.experimental.pallas import tpu as pltpu
from jax.experimental.pallas import tpu_sc as plsc

BATCH = 16384
HIDDEN = 64
OUT_DIM = 64

_NC = 2
_NS = 16
_NW = _NC * _NS
_NB = 7813
_CAP = 576
_ICH = 2048
_SENT = 500 << 21
_NBUF = 6
_PAD = BATCH + 64


def _gather_body(idx_hbm, tableT_hbm, out_hbm,
                 idx_c, selk_v, bands, rows_v,
                 off_s, gk_s, nb_s, sems, sem_out):
    wid = lax.axis_index("s") * _NC + lax.axis_index("c")
    lo = (_NB * wid) // _NW
    hi = (_NB * (wid + 1)) // _NW
    nloc = hi - lo
    iota = lax.iota(jnp.int32, 16)

    def round_body(carry):
        round_lo, _ = carry

        def sent(i, _):
            selk_v[pl.ds(i * 16, 16)] = jnp.full((16,), _SENT, jnp.int32)
            return 0
        lax.fori_loop(0, _CAP // 16, sent, 0)

        cnt = 0
        for ch in range(BATCH // _ICH):
            pltpu.sync_copy(idx_hbm.at[pl.ds(ch * _ICH, _ICH)], idx_c)

            def grp(g, cnt, ch=ch):
                v16 = idx_c[pl.ds(g * 16, 16)]
                rt16 = v16 >> 7
                lo16 = jnp.full((16,), lo, jnp.int32)
                hi16 = jnp.full((16,), hi, jnp.int32)
                one16 = jnp.full((16,), 1, jnp.int32)
                w16 = (rt16 - lo16) | (hi16 - one16 - rt16)
                mi = one16 - lax.shift_right_logical(w16, 31)
                c16 = plsc.cumsum(mi)
                ord16 = jnp.full((16,), cnt, jnp.int32) + c16 - one16
                rl16 = jnp.full((16,), round_lo, jnp.int32)
                cap16 = jnp.full((16,), _CAP, jnp.int32)
                win16 = (ord16 - rl16) | (rl16 + cap16 - one16 - ord16)
                mw = mi * (one16 - lax.shift_right_logical(win16, 31))
                k16 = (((rt16 - lo16) << 21) | ((v16 & 127) << 14)
                       | (iota + (ch * _ICH + g * 16)))
                dump16 = jnp.full((16,), _CAP + 8, jnp.int32)
                pos16 = (mw * (ord16 - rl16)
                         + (one16 - mw) * dump16)
                plsc.store_scatter(selk_v, [pos16], k16)
                return cnt + c16[15]

            cnt = lax.fori_loop(0, _ICH // 16, grp, cnt)

        def zero(b, _):
            off_s[b] = 0
            return 0
        lax.fori_loop(0, 256, zero, 0)

        def count(g, _):
            k16 = selk_v[pl.ds(g * 16, 16)]
            for l in range(16):
                b = k16[l] >> 21
                bb = jnp.where(b < nloc, b, 255)
                off_s[bb] = off_s[bb] + 1
            return 0
        lax.fori_loop(0, _CAP // 16, count, 0)

        def prefix(b, carry2):
            running, nb_cnt = carry2
            c = off_s[b]
            use = b < nloc

            @pl.when(use & (c > 0))
            def _():
                nb_s[nb_cnt] = b
            running = jnp.where(use, running + c, running)
            nb_cnt = jnp.where(use & (c > 0), nb_cnt + 1, nb_cnt)

            @pl.when(use)
            def _():
                off_s[b] = running
            return running, nb_cnt

        total, m_nb = lax.fori_loop(0, 246, prefix, (0, 0))
        off_s[246] = total

        def scat(j, _):
            g = (_CAP // 16 - 1) - j
            k16 = selk_v[pl.ds(g * 16, 16)]
            for l in reversed(range(16)):
                k = k16[l]
                b = k >> 21

                @pl.when(b < nloc)
                def _(k=k, b=b):
                    slot = off_s[b] - 1
                    off_s[b] = slot
                    gk_s[slot] = k
            return 0
        lax.fori_loop(0, _CAP // 16, scat, 0)

        def fetch(b_loc, buf, sem):
            pltpu.async_copy(
                tableT_hbm.at[:, pl.ds((lo + b_loc) * 128, 128)], buf, sem)

        for par in range(_NBUF):
            @pl.when(m_nb > par)
            def _(par=par):
                fetch(nb_s[par], bands[par], sems[par])

        def quad(u, _):
            for par in range(_NBUF):
                t = _NBUF * u + par
                buf = bands[par]
                sem = sems[par]

                @pl.when(t < m_nb)
                def _(t=t, buf=buf, sem=sem):
                    pltpu.make_async_copy(
                        tableT_hbm.at[:, pl.ds(0, 128)], buf, sem).wait()
                    b = nb_s[t]
                    start = off_s[b]
                    end = off_s[jnp.where(b + 1 < nloc, b + 1, 246)]
                    end = jnp.where(b + 1 < nloc, end, total)

                    def extract(e, _):
                        k = gk_s[e]
                        lane = (k >> 14) & 127
                        for c in range(HIDDEN // 16):
                            vals = plsc.load_gather(
                                buf, [iota + 16 * c,
                                      jnp.full((16,), lane, jnp.int32)])
                            plsc.store_scatter(
                                rows_v,
                                [jnp.full((16,), e, jnp.int32),
                                 iota + 16 * c],
                                vals)
                        return 0

                    lax.fori_loop(start, end, extract, 0)

                    @pl.when(t + _NBUF < m_nb)
                    def _():
                        fetch(nb_s[t + _NBUF], buf, sem)
            return 0

        lax.fori_loop(0, (246 + _NBUF - 1) // _NBUF, quad, 0)

        def flush(g, _):
            for l in range(16):
                e = g * 16 + l
                real = e < total
                pos = jnp.where(real, gk_s[jnp.where(real, e, 0)] & 16383,
                                BATCH + wid)
                pltpu.async_copy(rows_v.at[e], out_hbm.at[pos], sem_out)
            return 0

        lax.fori_loop(0, _CAP // 16, flush, 0)
        pltpu.make_async_copy(
            out_hbm.at[pl.ds(0, _CAP)], rows_v, sem_out).wait()

        return round_lo + _CAP, cnt

    lax.while_loop(lambda c: c[0] < c[1], round_body, (0, 1))


_gather = pl.kernel(
    _gather_body,
    out_type=jax.ShapeDtypeStruct((_PAD, HIDDEN), jnp.float32),
    mesh=plsc.VectorSubcoreMesh(core_axis_name="c", subcore_axis_name="s"),
    scratch_types=[
        pltpu.VMEM((_ICH,), jnp.int32),
        pltpu.VMEM((_CAP + 32,), jnp.int32),
        [pltpu.VMEM((HIDDEN, 128), jnp.float32) for _ in range(_NBUF)],
        pltpu.VMEM((_CAP, HIDDEN), jnp.float32),
        pltpu.SMEM((256,), jnp.int32),
        pltpu.SMEM((_CAP,), jnp.int32),
        pltpu.SMEM((256,), jnp.int32),
        [pltpu.SemaphoreType.DMA for _ in range(_NBUF)],
        pltpu.SemaphoreType.DMA,
    ],
    compiler_params=pltpu.CompilerParams(needs_layout_passes=False),
)

_BM = 2048


def _proj_body(x_ref, w_ref, b_ref, o_ref):
    o_ref[...] = (
        lax.dot_general(
            w_ref[...],
            x_ref[...],
            (((1,), (1,)), ((), ())),
            preferred_element_type=jnp.float32,
        )
        + b_ref[...]
    )


_project = pl.pallas_call(
    _proj_body,
    grid=(BATCH // _BM,),
    in_specs=[
        pl.BlockSpec((_BM, HIDDEN), lambda i: (i, 0)),
        pl.BlockSpec((OUT_DIM, HIDDEN), lambda i: (0, 0)),
        pl.BlockSpec((OUT_DIM, 1), lambda i: (0, 0)),
    ],
    out_specs=pl.BlockSpec((OUT_DIM, _BM), lambda i: (0, i)),
    out_shape=jax.ShapeDtypeStruct((OUT_DIM, BATCH), jnp.float32),
)


def kernel(course_id, embed_table, W, b):
    idx = course_id.reshape(BATCH).astype(jnp.int32)
    tableT = embed_table.T
    rows = _gather(idx, tableT)
    y = _project(rows, W, b.reshape(OUT_DIM, 1))
    return y.T.reshape(BATCH, 1, OUT_DIM)


# speedup vs baseline: 4.2383x; 1.0028x over previous
import jax
import jax.numpy as jnp
from jax import lax
from jax.experimental import pallas as pl
from jax.experimental.pallas import tpu as pltpu
from jax.experimental.pallas import tpu_sc as plsc

BATCH = 16384
HIDDEN = 64
OUT_DIM = 64

_NC = 2
_NS = 16
_NW = _NC * _NS
_NB = 7813
_CAP = 576
_ICH = 2048
_SENT = 500 << 21
_NBUF = 6
_PAD = BATCH + 64


def _gather_body(idx_hbm, tableT_hbm, out_hbm,
                 idx_c, selk_v, bands, rows_v,
                 off_s, gk_s, nb_s, sems, sem_out):
    wid = lax.axis_index("s") * _NC + lax.axis_index("c")
    lo = (_NB * wid) // _NW
    hi = (_NB * (wid + 1)) // _NW
    nloc = hi - lo
    iota = lax.iota(jnp.int32, 16)

    def round_body(carry):
        round_lo, _ = carry

        def sent(i, _):
            selk_v[pl.ds(i * 16, 16)] = jnp.full((16,), _SENT, jnp.int32)
            return 0
        lax.fori_loop(0, _CAP // 16, sent, 0)

        cnt = 0
        for ch in range(BATCH // _ICH):
            pltpu.sync_copy(idx_hbm.at[pl.ds(ch * _ICH, _ICH)], idx_c)

            def grp(g, cnt, ch=ch):
                v16 = idx_c[pl.ds(g * 16, 16)]
                rt16 = v16 >> 7
                lo16 = jnp.full((16,), lo, jnp.int32)
                hi16 = jnp.full((16,), hi, jnp.int32)
                one16 = jnp.full((16,), 1, jnp.int32)
                w16 = (rt16 - lo16) | (hi16 - one16 - rt16)
                mi = one16 - lax.shift_right_logical(w16, 31)
                c16 = plsc.cumsum(mi)
                ord16 = jnp.full((16,), cnt, jnp.int32) + c16 - one16
                rl16 = jnp.full((16,), round_lo, jnp.int32)
                cap16 = jnp.full((16,), _CAP, jnp.int32)
                win16 = (ord16 - rl16) | (rl16 + cap16 - one16 - ord16)
                mw = mi * (one16 - lax.shift_right_logical(win16, 31))
                k16 = (((rt16 - lo16) << 21) | ((v16 & 127) << 14)
                       | (iota + (ch * _ICH + g * 16)))
                dump16 = jnp.full((16,), _CAP + 8, jnp.int32)
                pos16 = (mw * (ord16 - rl16)
                         + (one16 - mw) * dump16)
                plsc.store_scatter(selk_v, [pos16], k16)
                return cnt + c16[15]

            cnt = lax.fori_loop(0, _ICH // 16, grp, cnt)

        def zero(b, _):
            off_s[b] = 0
            return 0
        lax.fori_loop(0, 256, zero, 0)

        def count(g, _):
            k16 = selk_v[pl.ds(g * 16, 16)]
            for l in range(16):
                b = k16[l] >> 21
                bb = jnp.where(b < nloc, b, 255)
                off_s[bb] = off_s[bb] + 1
            return 0
        lax.fori_loop(0, _CAP // 16, count, 0)

        def prefix(b, carry2):
            running, nb_cnt = carry2
            c = off_s[b]
            use = b < nloc

            @pl.when(use & (c > 0))
            def _():
                nb_s[nb_cnt] = b
            running = jnp.where(use, running + c, running)
            nb_cnt = jnp.where(use & (c > 0), nb_cnt + 1, nb_cnt)

            @pl.when(use)
            def _():
                off_s[b] = running
            return running, nb_cnt

        total, m_nb = lax.fori_loop(0, 246, prefix, (0, 0))
        off_s[246] = total

        def scat(j, _):
            g = (_CAP // 16 - 1) - j
            k16 = selk_v[pl.ds(g * 16, 16)]
            for l in reversed(range(16)):
                k = k16[l]
                b = k >> 21

                @pl.when(b < nloc)
                def _(k=k, b=b):
                    slot = off_s[b] - 1
                    off_s[b] = slot
                    gk_s[slot] = k
            return 0
        lax.fori_loop(0, _CAP // 16, scat, 0)

        def fetch(b_loc, buf, sem):
            pltpu.async_copy(
                tableT_hbm.at[:, pl.ds((lo + b_loc) * 128, 128)], buf, sem)

        for par in range(_NBUF):
            @pl.when(m_nb > par)
            def _(par=par):
                fetch(nb_s[par], bands[par], sems[par])

        def quad(u, _):
            for par in range(_NBUF):
                t = _NBUF * u + par
                buf = bands[par]
                sem = sems[par]

                @pl.when(t < m_nb)
                def _(t=t, buf=buf, sem=sem):
                    pltpu.make_async_copy(
                        tableT_hbm.at[:, pl.ds(0, 128)], buf, sem).wait()
                    b = nb_s[t]
                    start = off_s[b]
                    end = off_s[jnp.where(b + 1 < nloc, b + 1, 246)]
                    end = jnp.where(b + 1 < nloc, end, total)

                    def extract(e, _):
                        k = gk_s[e]
                        lane = (k >> 14) & 127
                        for c in range(HIDDEN // 16):
                            vals = plsc.load_gather(
                                buf, [iota + 16 * c,
                                      jnp.full((16,), lane, jnp.int32)])
                            plsc.store_scatter(
                                rows_v,
                                [jnp.full((16,), e, jnp.int32),
                                 iota + 16 * c],
                                vals)
                        return 0

                    lax.fori_loop(start, end, extract, 0)

                    @pl.when(t + _NBUF < m_nb)
                    def _():
                        fetch(nb_s[t + _NBUF], buf, sem)
            return 0

        lax.fori_loop(0, (246 + _NBUF - 1) // _NBUF, quad, 0)

        def flush(g, _):
            for l in range(16):
                e = g * 16 + l
                real = e < total
                pos = jnp.where(real, gk_s[jnp.where(real, e, 0)] & 16383,
                                BATCH + wid)
                pltpu.async_copy(rows_v.at[e], out_hbm.at[pos], sem_out)
            return 0

        lax.fori_loop(0, _CAP // 16, flush, 0)
        pltpu.make_async_copy(
            out_hbm.at[pl.ds(0, _CAP)], rows_v, sem_out).wait()

        return round_lo + _CAP, cnt

    lax.while_loop(lambda c: c[0] < c[1], round_body, (0, 1))


_gather = pl.kernel(
    _gather_body,
    out_type=jax.ShapeDtypeStruct((_PAD, HIDDEN), jnp.float32),
    mesh=plsc.VectorSubcoreMesh(core_axis_name="c", subcore_axis_name="s"),
    scratch_types=[
        pltpu.VMEM((_ICH,), jnp.int32),
        pltpu.VMEM((_CAP + 32,), jnp.int32),
        [pltpu.VMEM((HIDDEN, 128), jnp.float32) for _ in range(_NBUF)],
        pltpu.VMEM((_CAP, HIDDEN), jnp.float32),
        pltpu.SMEM((256,), jnp.int32),
        pltpu.SMEM((_CAP,), jnp.int32),
        pltpu.SMEM((256,), jnp.int32),
        [pltpu.SemaphoreType.DMA for _ in range(_NBUF)],
        pltpu.SemaphoreType.DMA,
    ],
    compiler_params=pltpu.CompilerParams(needs_layout_passes=False),
)

_BM = 2048


def _proj_body(x_ref, w_ref, b_ref, o_ref):
    o_ref[...] = (
        lax.dot_general(
            w_ref[...],
            x_ref[...],
            (((1,), (1,)), ((), ())),
            preferred_element_type=jnp.float32,
        )
        + b_ref[...]
    )


_project = pl.pallas_call(
    _proj_body,
    grid=(BATCH // _BM,),
    in_specs=[
        pl.BlockSpec((_BM, HIDDEN), lambda i: (i, 0)),
        pl.BlockSpec((OUT_DIM, HIDDEN), lambda i: (0, 0)),
        pl.BlockSpec((OUT_DIM, 1), lambda i: (0, 0)),
    ],
    out_specs=pl.BlockSpec((OUT_DIM, _BM), lambda i: (0, i)),
    out_shape=jax.ShapeDtypeStruct((OUT_DIM, BATCH), jnp.float32),
)


def kernel(course_id, embed_table, W, b):
    idx = course_id.reshape(BATCH).astype(jnp.int32)
    tableT = embed_table.T
    rows = _gather(idx, tableT)
    y = _project(rows, W, b.reshape(OUT_DIM, 1))
    return y.T.reshape(BATCH, 1, OUT_DIM)
